# Initial kernel scaffold; baseline (speedup 1.0000x reference)
import jax
import jax.numpy as jnp
from jax.experimental import pallas as pl


def kernel(pred, gt_acc, gt_steer, gt_reverse):
    raise NotImplementedError("write your pallas kernel here")


# speedup vs baseline: 5.1299x; 5.1299x over previous
import jax
import jax.numpy as jnp
from jax.experimental import pallas as pl
from jax.experimental.pallas import tpu as pltpu

_V = 204
_PAD = _V - 1
_HALF = (_V - 4) / 2.0
_SPLIT = 101

_B = 64
_N = 2048
_T3 = 3 * _N
_ROWS = 768
_GRID_T = _T3 // _ROWS


def _loss_kernel(x_ref, tgt_ref, out_ref):
    t = pl.program_id(1)
    x = x_ref[0]
    tgt = tgt_ref[0]

    col = jax.lax.broadcasted_iota(jnp.int32, (_ROWS, _V), 1)
    m = jnp.max(x, axis=1, keepdims=True)
    tok = jnp.min(jnp.where(x == m, col, _V), axis=1, keepdims=True)
    tokf = tok.astype(jnp.float32) / _HALF - 1.0

    e = jnp.exp(x - m)
    s_no = jnp.sum(jnp.where(col < _SPLIT, e, 0.0), axis=1, keepdims=True)
    s_yes = jnp.sum(jnp.where(col >= _SPLIT, e, 0.0), axis=1, keepdims=True)
    s_tot = jnp.sum(e, axis=1, keepdims=True)
    p_no = s_no / s_tot
    p_yes = s_yes / s_tot

    rm = jax.lax.broadcasted_iota(jnp.int32, (_ROWS, 1), 0) % 3
    pv = jnp.where(rm == 0, jnp.abs(tokf), tokf)
    d = pv - tgt
    ad = jnp.abs(d)
    sl = jnp.where(ad < 1.0, 0.5 * d * d, ad - 0.5)
    lse = jnp.logaddexp(p_no, p_yes)
    chosen = jnp.where(tgt == 0.0, p_no, p_yes)
    nll = lse - chosen
    valid = jnp.logical_and(rm == 2, tgt != float(_PAD)).astype(jnp.float32)

    zero = jnp.zeros_like(sl)
    upd = jnp.concatenate(
        [jnp.where(rm == 0, sl, zero),
         jnp.where(rm == 1, sl, zero),
         valid * nll,
         valid],
        axis=1)

    @pl.when(t == 0)
    def _():
        out_ref[0] = upd

    @pl.when(t != 0)
    def _():
        out_ref[0] += upd


def kernel(pred, gt_acc, gt_steer, gt_reverse):
    tgt = jnp.stack(
        [gt_acc, gt_steer, gt_reverse.astype(jnp.float32)], axis=-1
    ).reshape(_B, _T3, 1)

    out = pl.pallas_call(
        _loss_kernel,
        grid=(_B, _GRID_T),
        in_specs=[
            pl.BlockSpec((1, _ROWS, _V), lambda b, t: (b, t, 0)),
            pl.BlockSpec((1, _ROWS, 1), lambda b, t: (b, t, 0)),
        ],
        out_specs=pl.BlockSpec((1, _ROWS, 4), lambda b, t: (b, 0, 0)),
        out_shape=jax.ShapeDtypeStruct((_B, _ROWS, 4), jnp.float32),
        compiler_params=pltpu.CompilerParams(
            dimension_semantics=("parallel", "arbitrary")),
    )(pred, tgt)

    sums = jnp.sum(out, axis=(0, 1))
    acc_steer_val_loss = (sums[0] + sums[1]) / float(_B * _N)
    reverse_val_loss = sums[2] / jnp.maximum(sums[3], 1.0)
    return acc_steer_val_loss, reverse_val_loss


# speedup vs baseline: 6.0880x; 1.1868x over previous
import jax
import jax.numpy as jnp
import numpy as np
from jax.experimental import pallas as pl
from jax.experimental.pallas import tpu as pltpu

_V = 204
_PAD = _V - 1
_HALF = (_V - 4) / 2.0
_SPLIT = 101

_B = 64
_N = 2048
_T3 = 3 * _N
_ROWS = 1536
_SLABS = _ROWS // 128
_GRID_T = _T3 // _ROWS


def _loss_kernel(x_ref, tgt_ref, w_ref, out_ref):
    t = pl.program_id(1)
    x = x_ref[0]
    tgt = tgt_ref[0, 0]

    col = jax.lax.broadcasted_iota(jnp.int32, (_ROWS, _V), 1)
    m = jnp.max(x, axis=1, keepdims=True)
    tok = jnp.min(jnp.where(x == m, col, _V), axis=1, keepdims=True)
    e = jnp.exp(x - m)
    sums = jax.lax.dot(e, w_ref[...],
                       preferred_element_type=jnp.float32)

    tokf = tok.astype(jnp.float32).reshape(_SLABS, 128) / _HALF - 1.0
    s_no = sums[:, 0:1].reshape(_SLABS, 128)
    s_tot = sums[:, 1:2].reshape(_SLABS, 128)

    r = (jax.lax.broadcasted_iota(jnp.int32, (_SLABS, 128), 0) * 128
         + jax.lax.broadcasted_iota(jnp.int32, (_SLABS, 128), 1))
    rm = r % 3
    pv = jnp.where(rm == 0, jnp.abs(tokf), tokf)
    d = pv - tgt
    ad = jnp.abs(d)
    sl = jnp.where(ad < 1.0, 0.5 * d * d, ad - 0.5)
    inv = 1.0 / s_tot
    p_no = s_no * inv
    p_yes = (s_tot - s_no) * inv
    lse = jnp.logaddexp(p_no, p_yes)
    chosen = jnp.where(tgt == 0.0, p_no, p_yes)
    nll = lse - chosen
    valid = jnp.logical_and(rm == 2, tgt != float(_PAD)).astype(jnp.float32)

    zero = jnp.zeros_like(sl)
    upd = jnp.stack(
        [jnp.where(rm == 0, sl, zero),
         jnp.where(rm == 1, sl, zero),
         valid * nll,
         valid],
        axis=0)

    @pl.when(t == 0)
    def _():
        out_ref[0] = upd

    @pl.when(t != 0)
    def _():
        out_ref[0] += upd


def kernel(pred, gt_acc, gt_steer, gt_reverse):
    tgt = jnp.stack(
        [gt_acc, gt_steer, gt_reverse.astype(jnp.float32)], axis=-1
    ).reshape(_B, _GRID_T, _SLABS, 128)

    w = np.zeros((_V, 128), dtype=np.float32)
    w[:_SPLIT, 0] = 1.0
    w[:, 1] = 1.0
    w = jnp.asarray(w)

    out = pl.pallas_call(
        _loss_kernel,
        grid=(_B, _GRID_T),
        in_specs=[
            pl.BlockSpec((1, _ROWS, _V), lambda b, t: (b, t, 0)),
            pl.BlockSpec((1, 1, _SLABS, 128), lambda b, t: (b, t, 0, 0)),
            pl.BlockSpec((_V, 128), lambda b, t: (0, 0)),
        ],
        out_specs=pl.BlockSpec((1, 4, _SLABS, 128), lambda b, t: (b, 0, 0, 0)),
        out_shape=jax.ShapeDtypeStruct((_B, 4, _SLABS, 128), jnp.float32),
        compiler_params=pltpu.CompilerParams(
            dimension_semantics=("parallel", "arbitrary")),
    )(pred, tgt, w)

    sums = jnp.sum(out, axis=(0, 2, 3))
    acc_steer_val_loss = (sums[0] + sums[1]) / float(_B * _N)
    reverse_val_loss = sums[2] / jnp.maximum(sums[3], 1.0)
    return acc_steer_val_loss, reverse_val_loss
